# baseline (speedup 1.0000x reference)
import functools

import jax
import jax.numpy as jnp
from jax import lax
from jax.experimental import pallas as pl
from jax.experimental.pallas import tpu as pltpu
from jax.experimental.pallas import tpu_sc as plsc

N = 10000
E = 320000
D = 128
H = 128
C = 64

NC = 2
NS = 16
NW = NC * NS
CH = 128
BB = 16
CPW = -(-(-(-E // (NW * CH))) // BB) * BB
NBB = CPW // BB
E_PAD = NW * CPW * CH
NP = 10240
NSP = 10112
RPS = NSP // NS
BR = 1024
DUMP = NSP - 1
DW = 16
LASTR = N - (NS - 1) * RPS


def _mesh():
    return plsc.VectorSubcoreMesh(
        core_axis_name="c", subcore_axis_name="s", num_cores=NC, num_subcores=NS)


NGR = CPW // 4


@functools.lru_cache(maxsize=None)
def _make_segsum(ntab, with_deg=False):
    F = C
    NPAIR = CPW // 2
    scratch = [
        pltpu.VMEM((CPW, CH), jnp.int32),
        pltpu.VMEM((CPW, CH), jnp.int32),
        pltpu.VMEM((CH, F), jnp.float32),
        pltpu.VMEM((CH, F), jnp.float32),
        pltpu.VMEM_SHARED((NSP, F), jnp.float32),
        pltpu.VMEM_SHARED((NSP, F), jnp.float32),
        pltpu.SemaphoreType.DMA,
        pltpu.SemaphoreType.DMA,
    ]
    if with_deg:
        scratch += [
            pltpu.VMEM((CH, DW), jnp.float32),
            pltpu.VMEM_SHARED((NSP, DW), jnp.float32),
            pltpu.SemaphoreType.DMA,
        ]

    def body(*args):
        ys = args[:ntab]
        nin = ntab + 3 + (2 if with_deg else 0)
        src_hbm, dst_hbm, zrow_hbm = args[ntab:ntab + 3]
        if with_deg:
            z8_hbm, ones_hbm = args[ntab + 3:ntab + 5]
        outs = args[nin:nin + ntab + (1 if with_deg else 0)]
        if with_deg:
            ones_v, deg_sp, dsem = args[-3:]
            scr = args[nin + ntab + 1:-3]
        else:
            scr = args[nin + ntab:]
        (src_v, dst_v, b0, b1, tab_sp, acc_sp, g0, g1) = scr
        c = lax.axis_index("c")
        s = lax.axis_index("s")
        w = c * NS + s
        bufs = (b0, b1)
        gsem = (g0, g1)
        pltpu.sync_copy(src_hbm.at[w], src_v)
        pltpu.sync_copy(dst_hbm.at[w], dst_v)
        if with_deg:
            pltpu.sync_copy(ones_hbm, ones_v)
            pltpu.sync_copy(z8_hbm, deg_sp.at[pl.ds(s * RPS, RPS)])

        def deg_scat(j):
            pltpu.async_copy(ones_v, deg_sp.at[dst_v.at[j]], dsem, add=True)

        def wait_deg():
            pltpu.make_async_copy(ones_v, deg_sp.at[dst_v.at[0]], dsem).wait()

        for it, (y_hbm, acc_out) in enumerate(zip(ys, outs[:ntab])):
            dodeg = with_deg and it == 0
            @pl.when(s < NS - 1)
            def _():
                pltpu.sync_copy(y_hbm.at[pl.ds(s * RPS, RPS)],
                                tab_sp.at[pl.ds(s * RPS, RPS)])

            @pl.when(s == NS - 1)
            def _():
                pltpu.sync_copy(y_hbm.at[pl.ds((NS - 1) * RPS, LASTR)],
                                tab_sp.at[pl.ds((NS - 1) * RPS, LASTR)])

            pltpu.sync_copy(zrow_hbm, acc_sp.at[pl.ds(s * RPS, RPS)])
            plsc.subcore_barrier()

            def gath(j, b):
                pltpu.async_copy(tab_sp.at[src_v.at[j]], bufs[b], gsem[b])

            def wait_gath(j, b):
                pltpu.make_async_copy(tab_sp.at[src_v.at[j]], bufs[b],
                                      gsem[b]).wait()

            gath(0, 0)

            def pair(p, carry):
                j0 = 2 * p
                for k in range(2):
                    j = j0 + k
                    if k == 0:
                        gath(j0 + 1, 1)
                    else:
                        pl.when(p < NPAIR - 1)(lambda: gath(j0 + 2, 0))
                    if dodeg:
                        deg_scat(j)
                        if k == 0:
                            pl.when(p >= 1)(wait_deg)
                        else:
                            wait_deg()
                    wait_gath(j, k)
                    pltpu.sync_copy(bufs[k], acc_sp.at[dst_v.at[j]], add=True)
                return carry

            lax.fori_loop(0, NPAIR, pair, 0)
            if dodeg:
                wait_deg()
            plsc.subcore_barrier()
            pltpu.sync_copy(acc_sp.at[pl.ds(s * RPS, RPS)],
                            acc_out.at[c, pl.ds(s * RPS, RPS)])
            if dodeg:
                pltpu.sync_copy(deg_sp.at[pl.ds(s * RPS, RPS)],
                                outs[ntab].at[c, pl.ds(s * RPS, RPS)])
            plsc.subcore_barrier()

    out = [jax.ShapeDtypeStruct((NC, NSP, C), jnp.float32)] * ntab
    if with_deg:
        out.append(jax.ShapeDtypeStruct((NC, NSP, DW), jnp.float32))
    return pl.kernel(body, out_type=tuple(out) if len(out) > 1 else out[0],
                     mesh=_mesh(), scratch_types=scratch,
                     compiler_params=pltpu.CompilerParams(use_tc_tiling_on_sc=False))


@functools.lru_cache(maxsize=None)
def _make_deg():
    scratch = [
        pltpu.VMEM((BB, CH), jnp.int32),
        pltpu.VMEM((CH, DW), jnp.float32),
        pltpu.VMEM_SHARED((NSP, DW), jnp.float32),
    ]

    def body(dst_hbm, z8_hbm, ones_hbm, deg_out, dst_v, ones_v, deg_sp):
        c = lax.axis_index("c")
        s = lax.axis_index("s")
        w = c * NS + s
        pltpu.sync_copy(ones_hbm, ones_v)
        pltpu.sync_copy(z8_hbm, deg_sp.at[pl.ds(s * RPS, RPS)])
        plsc.subcore_barrier()

        def slab(bb, carry):
            pltpu.sync_copy(dst_hbm.at[w, pl.ds(bb * BB, BB)], dst_v)

            def chunk(j, carry2):
                pltpu.sync_copy(ones_v, deg_sp.at[dst_v.at[j]], add=True)
                return carry2

            return lax.fori_loop(0, BB, chunk, carry)

        lax.fori_loop(0, NBB, slab, 0)
        plsc.subcore_barrier()
        pltpu.sync_copy(deg_sp.at[pl.ds(s * RPS, RPS)],
                        deg_out.at[c, pl.ds(s * RPS, RPS)])

    return pl.kernel(body, out_type=jax.ShapeDtypeStruct((NC, NSP, DW), jnp.float32),
                     mesh=_mesh(), scratch_types=scratch,
                     compiler_params=pltpu.CompilerParams(use_tc_tiling_on_sc=False))


def _mm2_body(x_ref, wl_ref, wr_ref, b_ref, ya_ref, yb_ref, r_ref):
    xb = x_ref[...]
    y = jnp.dot(xb, wl_ref[...], preferred_element_type=jnp.float32)
    ya_ref[...] = y[:, :C]
    yb_ref[...] = y[:, C:]
    r_ref[...] = jnp.dot(xb, wr_ref[...],
                         preferred_element_type=jnp.float32) + b_ref[...]


def _stage_b_body(acca_ref, accb_ref, deg_ref, r1_ref, wl_ref, wr_ref, b_ref,
                  y2_ref, r2_ref):
    a = jnp.concatenate([acca_ref[0] + acca_ref[1],
                         accb_ref[0] + accb_ref[1]], axis=1)
    dg = deg_ref[0, :, 0:1] + deg_ref[1, :, 0:1]
    rd = 1.0 / jnp.maximum(dg, 1.0)
    h = jnp.maximum(a * rd + r1_ref[...], 0.0)
    y2_ref[...] = jnp.dot(h, wl_ref[...], preferred_element_type=jnp.float32)
    r2_ref[...] = jnp.dot(h, wr_ref[...],
                          preferred_element_type=jnp.float32) + b_ref[...]


def _stage_c_body(acc_ref, deg_ref, r2_ref, o_ref):
    a = acc_ref[0] + acc_ref[1]
    dg = deg_ref[0, :, 0:1] + deg_ref[1, :, 0:1]
    rd = 1.0 / jnp.maximum(dg, 1.0)
    o_ref[...] = a * rd + r2_ref[...]


def _full(shape):
    return pl.BlockSpec(shape, lambda i: (0,) * len(shape))


BRN = 1000


_stage_a = pl.pallas_call(
    _mm2_body,
    grid=(N // BRN,),
    in_specs=[pl.BlockSpec((BRN, D), lambda i: (i, 0)),
              _full((D, H)), _full((D, H)), _full((1, H))],
    out_specs=[pl.BlockSpec((BRN, C), lambda i: (i, 0)),
               pl.BlockSpec((BRN, C), lambda i: (i, 0)),
               pl.BlockSpec((BRN, H), lambda i: (i, 0))],
    out_shape=[jax.ShapeDtypeStruct((N, C), jnp.float32),
               jax.ShapeDtypeStruct((N, C), jnp.float32),
               jax.ShapeDtypeStruct((N, H), jnp.float32)],
)

_stage_b = pl.pallas_call(
    _stage_b_body,
    grid=(N // BRN,),
    in_specs=[pl.BlockSpec((NC, BRN, C), lambda i: (0, i, 0)),
              pl.BlockSpec((NC, BRN, C), lambda i: (0, i, 0)),
              pl.BlockSpec((NC, BRN, DW), lambda i: (0, i, 0)),
              pl.BlockSpec((BRN, H), lambda i: (i, 0)),
              _full((H, C)), _full((H, C)), _full((1, C))],
    out_specs=[pl.BlockSpec((BRN, C), lambda i: (i, 0)),
               pl.BlockSpec((BRN, C), lambda i: (i, 0))],
    out_shape=[jax.ShapeDtypeStruct((N, C), jnp.float32)] * 2,
)

_stage_c = pl.pallas_call(
    _stage_c_body,
    grid=(N // BRN,),
    in_specs=[pl.BlockSpec((NC, BRN, C), lambda i: (0, i, 0)),
              pl.BlockSpec((NC, BRN, DW), lambda i: (0, i, 0)),
              pl.BlockSpec((BRN, C), lambda i: (i, 0))],
    out_specs=pl.BlockSpec((BRN, C), lambda i: (i, 0)),
    out_shape=jax.ShapeDtypeStruct((N, C), jnp.float32),
)


def kernel(x, edge_index, W1_l, W1_r, b1, W2_l, W2_r, b2):
    src = edge_index[0].astype(jnp.int32)
    dst = edge_index[1].astype(jnp.int32)
    pad = E_PAD - E
    src_p = jnp.concatenate([src, jnp.zeros((pad,), jnp.int32)]
                            ).reshape(NW, CPW, CH)
    dst_p = jnp.concatenate([dst, jnp.full((pad,), DUMP, jnp.int32)]
                            ).reshape(NW, CPW, CH)
    zrow = jnp.zeros((RPS, C), jnp.float32)
    z8 = jnp.zeros((RPS, DW), jnp.float32)
    ones8 = jnp.ones((CH, DW), jnp.float32)

    y1a, y1b, r1 = _stage_a(x, W1_l, W1_r, b1.reshape(1, H))
    acc1a, acc1b, degp = _make_segsum(2, True)(y1a, y1b, src_p, dst_p, zrow,
                                               z8, ones8)
    y2, r2 = _stage_b(acc1a, acc1b, degp, r1, W2_l, W2_r, b2.reshape(1, C))
    acc2 = _make_segsum(1)(y2, src_p, dst_p, zrow)
    out = _stage_c(acc2, degp, r2)
    return (out, edge_index)


# speedup vs baseline: 1.0359x; 1.0359x over previous
import functools

import jax
import jax.numpy as jnp
from jax import lax
from jax.experimental import pallas as pl
from jax.experimental.pallas import tpu as pltpu
from jax.experimental.pallas import tpu_sc as plsc

N = 10000
E = 320000
D = 128
H = 128
C = 64

NC = 2
NS = 16
NW = NC * NS
CH = 128
BB = 16
CPW = -(-(-(-E // (NW * CH))) // BB) * BB
NBB = CPW // BB
E_PAD = NW * CPW * CH
NP = 10240
NSP = 10112
RPS = NSP // NS
BR = 1024
DUMP = NSP - 1
DW = 16
LASTR = N - (NS - 1) * RPS


def _mesh():
    return plsc.VectorSubcoreMesh(
        core_axis_name="c", subcore_axis_name="s", num_cores=NC, num_subcores=NS)


NGR = CPW // 4


@functools.lru_cache(maxsize=None)
def _make_segsum(ntab, with_deg=False):
    F = C
    NPAIR = CPW // 2
    scratch = [
        pltpu.VMEM((CPW, CH), jnp.int32),
        pltpu.VMEM((CPW, CH), jnp.int32),
        pltpu.VMEM((CH, F), jnp.float32),
        pltpu.VMEM((CH, F), jnp.float32),
        pltpu.VMEM_SHARED((NSP, F), jnp.float32),
        pltpu.VMEM_SHARED((NSP, F), jnp.float32),
        pltpu.SemaphoreType.DMA,
        pltpu.SemaphoreType.DMA,
    ]
    if with_deg:
        scratch += [
            pltpu.VMEM((CH, DW), jnp.float32),
            pltpu.VMEM_SHARED((NSP, DW), jnp.float32),
            pltpu.SemaphoreType.DMA,
        ]

    def body(*args):
        ys = args[:ntab]
        nin = ntab + 3 + (2 if with_deg else 0)
        src_hbm, dst_hbm, zrow_hbm = args[ntab:ntab + 3]
        if with_deg:
            z8_hbm, ones_hbm = args[ntab + 3:ntab + 5]
        outs = args[nin:nin + ntab + (1 if with_deg else 0)]
        if with_deg:
            ones_v, deg_sp, dsem = args[-3:]
            scr = args[nin + ntab + 1:-3]
        else:
            scr = args[nin + ntab:]
        (src_v, dst_v, b0, b1, tab_sp, acc_sp, g0, g1) = scr
        c = lax.axis_index("c")
        s = lax.axis_index("s")
        w = c * NS + s
        bufs = (b0, b1)
        gsem = (g0, g1)
        pltpu.sync_copy(src_hbm.at[w], src_v)
        pltpu.sync_copy(dst_hbm.at[w], dst_v)
        if with_deg:
            pltpu.sync_copy(ones_hbm, ones_v)
            pltpu.sync_copy(z8_hbm, deg_sp.at[pl.ds(s * RPS, RPS)])

        def deg_scat(j):
            pltpu.async_copy(ones_v, deg_sp.at[dst_v.at[j]], dsem, add=True)

        def wait_deg():
            pltpu.make_async_copy(ones_v, deg_sp.at[dst_v.at[0]], dsem).wait()

        for it, (y_hbm, acc_out) in enumerate(zip(ys, outs[:ntab])):
            dodeg = with_deg and it == 0
            @pl.when(s < NS - 1)
            def _():
                pltpu.sync_copy(y_hbm.at[pl.ds(s * RPS, RPS)],
                                tab_sp.at[pl.ds(s * RPS, RPS)])

            @pl.when(s == NS - 1)
            def _():
                pltpu.sync_copy(y_hbm.at[pl.ds((NS - 1) * RPS, LASTR)],
                                tab_sp.at[pl.ds((NS - 1) * RPS, LASTR)])

            pltpu.sync_copy(zrow_hbm, acc_sp.at[pl.ds(s * RPS, RPS)])
            plsc.subcore_barrier()

            def gath(j, b):
                pltpu.async_copy(tab_sp.at[src_v.at[j]], bufs[b], gsem[b])

            def wait_gath(j, b):
                pltpu.make_async_copy(tab_sp.at[src_v.at[j]], bufs[b],
                                      gsem[b]).wait()

            gath(0, 0)

            def pair(p, carry):
                j0 = 2 * p
                for k in range(2):
                    j = j0 + k
                    if k == 0:
                        gath(j0 + 1, 1)
                    else:
                        pl.when(p < NPAIR - 1)(lambda: gath(j0 + 2, 0))
                    if dodeg:
                        deg_scat(j)
                        if k == 0:
                            pl.when(p >= 1)(wait_deg)
                        else:
                            wait_deg()
                    wait_gath(j, k)
                    pltpu.sync_copy(bufs[k], acc_sp.at[dst_v.at[j]], add=True)
                return carry

            lax.fori_loop(0, NPAIR, pair, 0)
            if dodeg:
                wait_deg()
            plsc.subcore_barrier()
            pltpu.sync_copy(acc_sp.at[pl.ds(s * RPS, RPS)],
                            acc_out.at[c, pl.ds(s * RPS, RPS)])
            if dodeg:
                pltpu.sync_copy(deg_sp.at[pl.ds(s * RPS, RPS)],
                                outs[ntab].at[c, pl.ds(s * RPS, RPS)])
            plsc.subcore_barrier()

    out = [jax.ShapeDtypeStruct((NC, NSP, C), jnp.float32)] * ntab
    if with_deg:
        out.append(jax.ShapeDtypeStruct((NC, NSP, DW), jnp.float32))
    return pl.kernel(body, out_type=tuple(out) if len(out) > 1 else out[0],
                     mesh=_mesh(), scratch_types=scratch,
                     compiler_params=pltpu.CompilerParams(use_tc_tiling_on_sc=False))


@functools.lru_cache(maxsize=None)
def _make_deg():
    scratch = [
        pltpu.VMEM((CPW, CH), jnp.int32),
        pltpu.VMEM((CH, DW), jnp.float32),
        pltpu.VMEM_SHARED((NSP, DW), jnp.float32),
        pltpu.SemaphoreType.DMA,
    ]

    def body(dst_hbm, z8_hbm, ones_hbm, deg_out, dst_v, ones_v, deg_sp, dsem):
        c = lax.axis_index("c")
        s = lax.axis_index("s")
        w = c * NS + s
        pltpu.sync_copy(ones_hbm, ones_v)
        pltpu.sync_copy(dst_hbm.at[w], dst_v)
        pltpu.sync_copy(z8_hbm, deg_sp.at[pl.ds(s * RPS, RPS)])
        plsc.subcore_barrier()

        def fire(j):
            pltpu.async_copy(ones_v, deg_sp.at[dst_v.at[j]], dsem, add=True)

        def drain():
            pltpu.make_async_copy(ones_v, deg_sp.at[dst_v.at[0]], dsem).wait()

        fire(0)
        fire(1)

        def chunk(j, carry):
            drain()
            fire(j)
            return carry

        lax.fori_loop(2, CPW, chunk, 0)
        drain()
        drain()
        plsc.subcore_barrier()
        pltpu.sync_copy(deg_sp.at[pl.ds(s * RPS, RPS)],
                        deg_out.at[c, pl.ds(s * RPS, RPS)])

    return pl.kernel(body, out_type=jax.ShapeDtypeStruct((NC, NSP, DW), jnp.float32),
                     mesh=_mesh(), scratch_types=scratch,
                     compiler_params=pltpu.CompilerParams(use_tc_tiling_on_sc=False))


def _mm2_body(x_ref, wl_ref, wr_ref, b_ref, ya_ref, yb_ref, r_ref):
    xb = x_ref[...]
    y = jnp.dot(xb, wl_ref[...], preferred_element_type=jnp.float32)
    ya_ref[...] = y[:, :C]
    yb_ref[...] = y[:, C:]
    r_ref[...] = jnp.dot(xb, wr_ref[...],
                         preferred_element_type=jnp.float32) + b_ref[...]


def _stage_b_body(acca_ref, accb_ref, deg_ref, r1_ref, wl_ref, wr_ref, b_ref,
                  y2_ref, r2_ref):
    a = jnp.concatenate([acca_ref[0] + acca_ref[1],
                         accb_ref[0] + accb_ref[1]], axis=1)
    dg = deg_ref[0, :, 0:1] + deg_ref[1, :, 0:1]
    rd = 1.0 / jnp.maximum(dg, 1.0)
    h = jnp.maximum(a * rd + r1_ref[...], 0.0)
    y2_ref[...] = jnp.dot(h, wl_ref[...], preferred_element_type=jnp.float32)
    r2_ref[...] = jnp.dot(h, wr_ref[...],
                          preferred_element_type=jnp.float32) + b_ref[...]


def _stage_c_body(acc_ref, deg_ref, r2_ref, o_ref):
    a = acc_ref[0] + acc_ref[1]
    dg = deg_ref[0, :, 0:1] + deg_ref[1, :, 0:1]
    rd = 1.0 / jnp.maximum(dg, 1.0)
    o_ref[...] = a * rd + r2_ref[...]


def _full(shape):
    return pl.BlockSpec(shape, lambda i: (0,) * len(shape))


BRN = 1000


_stage_a = pl.pallas_call(
    _mm2_body,
    grid=(N // BRN,),
    in_specs=[pl.BlockSpec((BRN, D), lambda i: (i, 0)),
              _full((D, H)), _full((D, H)), _full((1, H))],
    out_specs=[pl.BlockSpec((BRN, C), lambda i: (i, 0)),
               pl.BlockSpec((BRN, C), lambda i: (i, 0)),
               pl.BlockSpec((BRN, H), lambda i: (i, 0))],
    out_shape=[jax.ShapeDtypeStruct((N, C), jnp.float32),
               jax.ShapeDtypeStruct((N, C), jnp.float32),
               jax.ShapeDtypeStruct((N, H), jnp.float32)],
)

_stage_b = pl.pallas_call(
    _stage_b_body,
    grid=(N // BRN,),
    in_specs=[pl.BlockSpec((NC, BRN, C), lambda i: (0, i, 0)),
              pl.BlockSpec((NC, BRN, C), lambda i: (0, i, 0)),
              pl.BlockSpec((NC, BRN, DW), lambda i: (0, i, 0)),
              pl.BlockSpec((BRN, H), lambda i: (i, 0)),
              _full((H, C)), _full((H, C)), _full((1, C))],
    out_specs=[pl.BlockSpec((BRN, C), lambda i: (i, 0)),
               pl.BlockSpec((BRN, C), lambda i: (i, 0))],
    out_shape=[jax.ShapeDtypeStruct((N, C), jnp.float32)] * 2,
)

_stage_c = pl.pallas_call(
    _stage_c_body,
    grid=(N // BRN,),
    in_specs=[pl.BlockSpec((NC, BRN, C), lambda i: (0, i, 0)),
              pl.BlockSpec((NC, BRN, DW), lambda i: (0, i, 0)),
              pl.BlockSpec((BRN, C), lambda i: (i, 0))],
    out_specs=pl.BlockSpec((BRN, C), lambda i: (i, 0)),
    out_shape=jax.ShapeDtypeStruct((N, C), jnp.float32),
)


def kernel(x, edge_index, W1_l, W1_r, b1, W2_l, W2_r, b2):
    src = edge_index[0].astype(jnp.int32)
    dst = edge_index[1].astype(jnp.int32)
    pad = E_PAD - E
    src_p = jnp.concatenate([src, jnp.zeros((pad,), jnp.int32)]
                            ).reshape(NW, CPW, CH)
    dst_p = jnp.concatenate([dst, jnp.full((pad,), DUMP, jnp.int32)]
                            ).reshape(NW, CPW, CH)
    zrow = jnp.zeros((RPS, C), jnp.float32)
    z8 = jnp.zeros((RPS, DW), jnp.float32)
    ones8 = jnp.ones((CH, DW), jnp.float32)

    degp = _make_deg()(dst_p, z8, ones8)
    y1a, y1b, r1 = _stage_a(x, W1_l, W1_r, b1.reshape(1, H))
    acc1a, acc1b = _make_segsum(2)(y1a, y1b, src_p, dst_p, zrow)
    y2, r2 = _stage_b(acc1a, acc1b, degp, r1, W2_l, W2_r, b2.reshape(1, C))
    acc2 = _make_segsum(1)(y2, src_p, dst_p, zrow)
    out = _stage_c(acc2, degp, r2)
    return (out, edge_index)
